# Initial kernel scaffold; baseline (speedup 1.0000x reference)
import jax
import jax.numpy as jnp
from jax.experimental import pallas as pl


def kernel(x, Wb, bb, Wdisc, Wdown, Wup, mapping):
    raise NotImplementedError("write your pallas kernel here")


# speedup vs baseline: 1.1042x; 1.1042x over previous
import functools

import jax
import jax.numpy as jnp
from jax.experimental import pallas as pl
from jax.experimental.pallas import tpu as pltpu


def _route_body(x_ref, wdisc_ref, mapping_ref, aidx_ref, acc_ref):
    t = pl.program_id(1)

    @pl.when(t == 0)
    def _():
        acc_ref[...] = jnp.zeros_like(acc_ref)

    xb = x_ref[0]
    proj = jnp.dot(xb, wdisc_ref[...].T, preferred_element_type=jnp.float32)
    acc_ref[...] += jnp.sum(proj * proj, axis=0, keepdims=True)

    @pl.when(t == pl.num_programs(1) - 1)
    def _():
        top1 = jnp.argmin(acc_ref[0], axis=0)
        aidx_ref[pl.program_id(0)] = mapping_ref[top1]


def _main_body(aidx_ref, x_ref, wb_ref, bb_ref, wd_ref, wu_ref, out_ref):
    xb = x_ref[0]
    base = jnp.dot(xb, wb_ref[...], preferred_element_type=jnp.float32)
    base = base + bb_ref[...]
    h = jnp.maximum(jnp.dot(xb, wd_ref[0], preferred_element_type=jnp.float32), 0.0)
    out_ref[0] = base + jnp.dot(h, wu_ref[0], preferred_element_type=jnp.float32)


@functools.partial(jax.jit, static_argnames=("interpret",))
def _run(x, Wb, bb, Wdisc, Wdown, Wup, mapping, interpret=False):
    B, T, D = x.shape
    E, _, R = Wdown.shape
    TT = 512
    Tn = T // TT

    aidx = pl.pallas_call(
        _route_body,
        grid=(B, Tn),
        in_specs=[
            pl.BlockSpec((1, TT, D), lambda b, t: (b, t, 0)),
            pl.BlockSpec((E, D), lambda b, t: (0, 0)),
            pl.BlockSpec(memory_space=pltpu.SMEM),
        ],
        out_specs=pl.BlockSpec((B,), lambda b, t: (0,), memory_space=pltpu.SMEM),
        out_shape=jax.ShapeDtypeStruct((B,), jnp.int32),
        scratch_shapes=[pltpu.VMEM((1, E), jnp.float32)],
        interpret=interpret,
    )(x, Wdisc, mapping)

    out = pl.pallas_call(
        _main_body,
        grid_spec=pltpu.PrefetchScalarGridSpec(
            num_scalar_prefetch=1,
            grid=(B, Tn),
            in_specs=[
                pl.BlockSpec((1, TT, D), lambda b, t, aidx: (b, t, 0)),
                pl.BlockSpec((D, D), lambda b, t, aidx: (0, 0)),
                pl.BlockSpec((1, D), lambda b, t, aidx: (0, 0)),
                pl.BlockSpec((1, D, R), lambda b, t, aidx: (aidx[b], 0, 0)),
                pl.BlockSpec((1, R, D), lambda b, t, aidx: (aidx[b], 0, 0)),
            ],
            out_specs=pl.BlockSpec((1, TT, D), lambda b, t, aidx: (b, t, 0)),
        ),
        out_shape=jax.ShapeDtypeStruct((B, T, D), jnp.float32),
        interpret=interpret,
    )(aidx, x, Wb, bb.reshape(1, D), Wdown, Wup)
    return out


def kernel(x, Wb, bb, Wdisc, Wdown, Wup, mapping):
    return _run(x, Wb, bb, Wdisc, Wdown, Wup, mapping)


# speedup vs baseline: 1.2854x; 1.1641x over previous
import functools

import jax
import jax.numpy as jnp
from jax.experimental import pallas as pl
from jax.experimental.pallas import tpu as pltpu


def _route_body(x_ref, wdisc_ref, mapping_ref, aidx_ref, acc_ref):
    t = pl.program_id(1)

    @pl.when(t == 0)
    def _():
        acc_ref[...] = jnp.zeros_like(acc_ref)

    xb = x_ref[0]
    proj = jnp.dot(xb, wdisc_ref[...].T, preferred_element_type=jnp.float32)
    acc_ref[...] += jnp.sum(proj * proj, axis=0, keepdims=True)

    @pl.when(t == pl.num_programs(1) - 1)
    def _():
        top1 = jnp.argmin(acc_ref[0], axis=0)
        aidx_ref[pl.program_id(0)] = mapping_ref[top1]


def _main_body(aidx_ref, x_ref, wb_ref, bb_ref, wd_ref, wu_ref, out_ref):
    xb = x_ref[0]
    base = jnp.dot(xb, wb_ref[...], preferred_element_type=jnp.float32)
    base = base + bb_ref[...]
    h = jnp.maximum(jnp.dot(xb, wd_ref[0], preferred_element_type=jnp.float32), 0.0)
    out_ref[0] = base + jnp.dot(h, wu_ref[0], preferred_element_type=jnp.float32)


@functools.partial(jax.jit, static_argnames=("interpret",))
def _run(x, Wb, bb, Wdisc, Wdown, Wup, mapping, interpret=False):
    B, T, D = x.shape
    E, _, R = Wdown.shape
    TT = 1024
    Tn = T // TT

    aidx = pl.pallas_call(
        _route_body,
        grid=(B, Tn),
        in_specs=[
            pl.BlockSpec((1, TT, D), lambda b, t: (b, t, 0)),
            pl.BlockSpec((E, D), lambda b, t: (0, 0)),
            pl.BlockSpec(memory_space=pltpu.SMEM),
        ],
        out_specs=pl.BlockSpec((B,), lambda b, t: (0,), memory_space=pltpu.SMEM),
        out_shape=jax.ShapeDtypeStruct((B,), jnp.int32),
        scratch_shapes=[pltpu.VMEM((1, E), jnp.float32)],
        compiler_params=pltpu.CompilerParams(
            dimension_semantics=("arbitrary", "arbitrary")),
        interpret=interpret,
    )(x, Wdisc, mapping)

    out = pl.pallas_call(
        _main_body,
        grid_spec=pltpu.PrefetchScalarGridSpec(
            num_scalar_prefetch=1,
            grid=(B, Tn),
            in_specs=[
                pl.BlockSpec((1, TT, D), lambda b, t, aidx: (b, t, 0)),
                pl.BlockSpec((D, D), lambda b, t, aidx: (0, 0)),
                pl.BlockSpec((1, D), lambda b, t, aidx: (0, 0)),
                pl.BlockSpec((1, D, R), lambda b, t, aidx: (aidx[b], 0, 0)),
                pl.BlockSpec((1, R, D), lambda b, t, aidx: (aidx[b], 0, 0)),
            ],
            out_specs=pl.BlockSpec((1, TT, D), lambda b, t, aidx: (b, t, 0)),
        ),
        out_shape=jax.ShapeDtypeStruct((B, T, D), jnp.float32),
        compiler_params=pltpu.CompilerParams(
            dimension_semantics=("parallel", "parallel")),
        interpret=interpret,
    )(aidx, x, Wb, bb.reshape(1, D), Wdown, Wup)
    return out


def kernel(x, Wb, bb, Wdisc, Wdown, Wup, mapping):
    return _run(x, Wb, bb, Wdisc, Wdown, Wup, mapping)


# speedup vs baseline: 1.3421x; 1.0441x over previous
import functools

import jax
import jax.numpy as jnp
from jax.experimental import pallas as pl
from jax.experimental.pallas import tpu as pltpu


def _route_body(x_ref, wdisc_ref, mapping_ref, aidx_ref, acc_ref):
    t = pl.program_id(1)

    @pl.when(t == 0)
    def _():
        acc_ref[...] = jnp.zeros_like(acc_ref)

    xb = x_ref[0]
    proj = jnp.dot(xb, wdisc_ref[...].T, preferred_element_type=jnp.float32)
    acc_ref[...] += jnp.sum(proj * proj, axis=0, keepdims=True)

    @pl.when(t == pl.num_programs(1) - 1)
    def _():
        top1 = jnp.argmin(acc_ref[0], axis=0)
        aidx_ref[pl.program_id(0)] = mapping_ref[top1]


def _main_body(aidx_ref, x_ref, wb_ref, bb_ref, wd_ref, wu_ref, out_ref):
    xb = x_ref[0]
    base = jnp.dot(xb, wb_ref[...], preferred_element_type=jnp.float32)
    base = base + bb_ref[...]
    h = jnp.maximum(jnp.dot(xb, wd_ref[0], preferred_element_type=jnp.float32), 0.0)
    out_ref[0] = base + jnp.dot(h, wu_ref[0], preferred_element_type=jnp.float32)


@functools.partial(jax.jit, static_argnames=("interpret",))
def _run(x, Wb, bb, Wdisc, Wdown, Wup, mapping, interpret=False):
    B, T, D = x.shape
    E, _, R = Wdown.shape
    TT = 2048
    Tn = T // TT

    aidx = pl.pallas_call(
        _route_body,
        grid=(B, Tn),
        in_specs=[
            pl.BlockSpec((1, TT, D), lambda b, t: (b, t, 0)),
            pl.BlockSpec((E, D), lambda b, t: (0, 0)),
            pl.BlockSpec(memory_space=pltpu.SMEM),
        ],
        out_specs=pl.BlockSpec((B,), lambda b, t: (0,), memory_space=pltpu.SMEM),
        out_shape=jax.ShapeDtypeStruct((B,), jnp.int32),
        scratch_shapes=[pltpu.VMEM((1, E), jnp.float32)],
        compiler_params=pltpu.CompilerParams(
            dimension_semantics=("arbitrary", "arbitrary")),
        interpret=interpret,
    )(x, Wdisc, mapping)

    out = pl.pallas_call(
        _main_body,
        grid_spec=pltpu.PrefetchScalarGridSpec(
            num_scalar_prefetch=1,
            grid=(B, Tn),
            in_specs=[
                pl.BlockSpec((1, TT, D), lambda b, t, aidx: (b, t, 0)),
                pl.BlockSpec((D, D), lambda b, t, aidx: (0, 0)),
                pl.BlockSpec((1, D), lambda b, t, aidx: (0, 0)),
                pl.BlockSpec((1, D, R), lambda b, t, aidx: (aidx[b], 0, 0)),
                pl.BlockSpec((1, R, D), lambda b, t, aidx: (aidx[b], 0, 0)),
            ],
            out_specs=pl.BlockSpec((1, TT, D), lambda b, t, aidx: (b, t, 0)),
        ),
        out_shape=jax.ShapeDtypeStruct((B, T, D), jnp.float32),
        compiler_params=pltpu.CompilerParams(
            dimension_semantics=("parallel", "parallel")),
        interpret=interpret,
    )(aidx, x, Wb, bb.reshape(1, D), Wdown, Wup)
    return out


def kernel(x, Wb, bb, Wdisc, Wdown, Wup, mapping):
    return _run(x, Wb, bb, Wdisc, Wdown, Wup, mapping)
